# Initial kernel scaffold; baseline (speedup 1.0000x reference)
import jax
import jax.numpy as jnp
from jax.experimental import pallas as pl


def kernel(hidden_states, Wr, w1, w3, w2):
    raise NotImplementedError("write your pallas kernel here")


# speedup vs baseline: 1.3450x; 1.3450x over previous
import functools

import jax
import jax.numpy as jnp
from jax import lax
from jax.experimental import pallas as pl
from jax.experimental.pallas import tpu as pltpu
from jax.experimental.pallas import tpu_sc as plsc

NE = 8
TOPK = 2
DM = 1024
DFF = 2048
S = 2048
NA = S * TOPK

T = 128
NBLK = 40
NSLOT = NBLK * T

NW = 32
TPW = S // NW
CH = 512
NCH = NA // CH


def _router_body(x_ref, wr_ref, sel_ref, w_ref):
    x = x_ref[...]
    logits = jnp.dot(x, wr_ref[...], preferred_element_type=jnp.float32)
    m = jnp.max(logits, axis=1, keepdims=True)
    e = jnp.exp(logits - m)
    probs = e / jnp.sum(e, axis=1, keepdims=True)
    idx = lax.broadcasted_iota(jnp.int32, (S, NE), 1)
    m1 = jnp.max(probs, axis=1, keepdims=True)
    a1 = jnp.min(jnp.where(probs == m1, idx, NE), axis=1, keepdims=True)
    masked = jnp.where(idx == a1, -1.0, probs)
    m2 = jnp.max(masked, axis=1, keepdims=True)
    a2 = jnp.min(jnp.where(masked == m2, idx, NE), axis=1, keepdims=True)
    s = m1 + m2
    sel_ref[...] = jnp.concatenate([a1, a2], axis=1)
    w_ref[...] = jnp.concatenate([m1 / s, m2 / s], axis=1)


def _router(x, Wr):
    return pl.pallas_call(
        _router_body,
        out_shape=[
            jax.ShapeDtypeStruct((S, TOPK), jnp.int32),
            jax.ShapeDtypeStruct((S, TOPK), jnp.float32),
        ],
    )(x, Wr)


def _dispatch(sel_flat):
    mesh = plsc.ScalarSubcoreMesh(axis_name="c", num_cores=2)

    @functools.partial(
        pl.kernel,
        mesh=mesh,
        out_type=[
            jax.ShapeDtypeStruct((S,), jnp.int32),
            jax.ShapeDtypeStruct((S,), jnp.int32),
            jax.ShapeDtypeStruct((NBLK,), jnp.int32),
        ],
        scratch_types=[
            pltpu.SMEM((CH,), jnp.int32),
            pltpu.SMEM((CH // 2,), jnp.int32),
            pltpu.SMEM((CH // 2,), jnp.int32),
            pltpu.SMEM((NE,), jnp.int32),
            pltpu.SMEM((NE,), jnp.int32),
            pltpu.SMEM((NBLK,), jnp.int32),
            pltpu.SMEM((2,), jnp.int32),
            pltpu.SemaphoreType.DMA,
        ],
    )
    def k(sel_hbm, pos0_hbm, pos1_hbm, be_hbm,
          selb, p0b, p1b, cnt, start, beb, accum, sem):
        @pl.when(lax.axis_index("c") == 0)
        def _():
            @pl.loop(0, NE)
            def _(i):
                cnt[i] = 0

            @pl.loop(0, NCH)
            def _(c):
                pltpu.async_copy(sel_hbm.at[pl.ds(c * CH, CH)], selb, sem).wait()

                @pl.loop(0, CH)
                def _(i):
                    ei = selb[i]
                    cnt[ei] = cnt[ei] + 1

            accum[0] = 0
            accum[1] = 0

            @pl.loop(0, NE)
            def _(ei):
                start[ei] = accum[0]
                nb = (cnt[ei] + (T - 1)) // T
                bbase = accum[1]

                @pl.loop(0, NBLK)
                def _(b):
                    @pl.when(b < nb)
                    def _():
                        beb[bbase + b] = ei

                accum[0] = accum[0] + nb * T
                accum[1] = accum[1] + nb

            @pl.loop(0, NBLK)
            def _(b):
                @pl.when(b >= accum[1])
                def _():
                    beb[b] = beb[accum[1] - 1]

            @pl.loop(0, NE)
            def _(i):
                cnt[i] = 0

            @pl.loop(0, NCH)
            def _(c):
                pltpu.async_copy(sel_hbm.at[pl.ds(c * CH, CH)], selb, sem).wait()

                @pl.loop(0, CH // 2)
                def _(i):
                    e0 = selb[2 * i]
                    p0b[i] = start[e0] + cnt[e0]
                    cnt[e0] = cnt[e0] + 1
                    e1 = selb[2 * i + 1]
                    p1b[i] = start[e1] + cnt[e1]
                    cnt[e1] = cnt[e1] + 1

                half = CH // 2
                pltpu.async_copy(p0b, pos0_hbm.at[pl.ds(c * half, half)], sem).wait()
                pltpu.async_copy(p1b, pos1_hbm.at[pl.ds(c * half, half)], sem).wait()

            pltpu.async_copy(beb, be_hbm, sem).wait()

    return k(sel_flat)


def _scatter(x, pos0, pos1):
    mesh = plsc.VectorSubcoreMesh(core_axis_name="c", subcore_axis_name="s")

    @functools.partial(
        pl.kernel,
        mesh=mesh,
        out_type=jax.ShapeDtypeStruct((NSLOT, DM), jnp.float32),
        scratch_types=[
            pltpu.VMEM((TPW,), jnp.int32),
            pltpu.VMEM((TPW,), jnp.int32),
            pltpu.VMEM((TPW, DM), jnp.float32),
            pltpu.SemaphoreType.DMA,
        ],
    )
    def k(x_hbm, p0_hbm, p1_hbm, xg_hbm, i0v, i1v, rows, sem):
        wid = lax.axis_index("s") * 2 + lax.axis_index("c")
        base = wid * TPW
        pltpu.sync_copy(p0_hbm.at[pl.ds(base, TPW)], i0v)
        pltpu.sync_copy(p1_hbm.at[pl.ds(base, TPW)], i1v)
        pltpu.sync_copy(x_hbm.at[pl.ds(base, TPW)], rows)
        pltpu.async_copy(rows, xg_hbm.at[i0v], sem).wait()
        pltpu.async_copy(rows, xg_hbm.at[i1v], sem).wait()

    return k(x, pos0, pos1)


def _ffn_body(be_ref, xg_ref, w1_ref, w3_ref, w2_ref, o_ref):
    x = xg_ref[...]
    acc = jnp.zeros((T, DM), jnp.float32)
    for j in range(2):
        sl = slice(j * (DFF // 2), (j + 1) * (DFF // 2))
        a = jnp.dot(x, w1_ref[0, :, sl], preferred_element_type=jnp.float32)
        b = jnp.dot(x, w3_ref[0, :, sl], preferred_element_type=jnp.float32)
        h = a * (1.0 / (1.0 + jnp.exp(-a))) * b
        acc = acc + jnp.dot(h, w2_ref[0, sl, :], preferred_element_type=jnp.float32)
    o_ref[...] = acc


def _ffn(be, xg, w1, w3, w2):
    grid_spec = pltpu.PrefetchScalarGridSpec(
        num_scalar_prefetch=1,
        grid=(NBLK,),
        in_specs=[
            pl.BlockSpec((T, DM), lambda b, be: (b, 0)),
            pl.BlockSpec((1, DM, DFF), lambda b, be: (be[b], 0, 0)),
            pl.BlockSpec((1, DM, DFF), lambda b, be: (be[b], 0, 0)),
            pl.BlockSpec((1, DFF, DM), lambda b, be: (be[b], 0, 0)),
        ],
        out_specs=pl.BlockSpec((T, DM), lambda b, be: (b, 0)),
    )
    return pl.pallas_call(
        _ffn_body,
        grid_spec=grid_spec,
        out_shape=jax.ShapeDtypeStruct((NSLOT, DM), jnp.float32),
    )(be, xg, w1, w3, w2)


def _gather(os_, pos0, pos1):
    mesh = plsc.VectorSubcoreMesh(core_axis_name="c", subcore_axis_name="s")

    @functools.partial(
        pl.kernel,
        mesh=mesh,
        out_type=[
            jax.ShapeDtypeStruct((S, DM), jnp.float32),
            jax.ShapeDtypeStruct((S, DM), jnp.float32),
        ],
        scratch_types=[
            pltpu.VMEM((TPW,), jnp.int32),
            pltpu.VMEM((TPW,), jnp.int32),
            pltpu.VMEM((TPW, DM), jnp.float32),
            pltpu.SemaphoreType.DMA,
        ],
    )
    def k(os_hbm, p0_hbm, p1_hbm, g0_hbm, g1_hbm, i0v, i1v, rows, sem):
        wid = lax.axis_index("s") * 2 + lax.axis_index("c")
        base = wid * TPW
        pltpu.sync_copy(p0_hbm.at[pl.ds(base, TPW)], i0v)
        pltpu.sync_copy(p1_hbm.at[pl.ds(base, TPW)], i1v)
        pltpu.async_copy(os_hbm.at[i0v], rows, sem).wait()
        pltpu.sync_copy(rows, g0_hbm.at[pl.ds(base, TPW)])
        pltpu.async_copy(os_hbm.at[i1v], rows, sem).wait()
        pltpu.sync_copy(rows, g1_hbm.at[pl.ds(base, TPW)])

    return k(os_, pos0, pos1)


def _combine_body(g0_ref, g1_ref, w_ref, o_ref):
    w = w_ref[...]
    o_ref[...] = g0_ref[...] * w[:, 0:1] + g1_ref[...] * w[:, 1:2]


def _combine(g0, g1, w):
    return pl.pallas_call(
        _combine_body,
        out_shape=jax.ShapeDtypeStruct((S, DM), jnp.float32),
    )(g0, g1, w)


def kernel(hidden_states, Wr, w1, w3, w2):
    x = hidden_states.reshape(S, DM)
    sel, w = _router(x, Wr)
    pos0, pos1, be = _dispatch(sel.reshape(NA))
    xg = _scatter(x, pos0, pos1)
    os_ = _ffn(be, xg, w1, w3, w2)
    g0, g1 = _gather(os_, pos0, pos1)
    out = _combine(g0, g1, w)
    return out.reshape(1, S, DM)
